# baseline (speedup 1.0000x reference)
import functools
import math

import jax
import jax.numpy as jnp
from jax import lax
from jax.experimental import pallas as pl
from jax.experimental.pallas import tpu as pltpu
from jax.experimental.pallas import tpu_sc as plsc

S = 4096
E = 8
NC = 2
NS = 16
NW = NC * NS
CHUNK = S // NW
G = CHUNK // 16


def _cap(s, e):
    c = math.floor(1.0 * s / e)
    c += c % 2
    return max(c, 4)


CAP = _cap(S, E)
F = E * CAP


def _route_body(x_hbm, idx_hbm, lrank_hbm, prob_hbm, counts_hbm,
                xv, idxv, lrankv, probv, cntv):
    c = lax.axis_index("c")
    s = lax.axis_index("s")
    wid = s * NC + c
    t0 = wid * CHUNK
    pltpu.sync_copy(x_hbm.at[pl.ds(t0 * E, CHUNK * E)], xv)

    lane = jnp.arange(16, dtype=jnp.int32)
    for g in range(G):
        base = (lane + g * 16) * E
        xs = [plsc.load_gather(xv, [base + e]) for e in range(E)]
        m = xs[0]
        idx = jnp.zeros((16,), jnp.int32)
        for e in range(1, E):
            gt = xs[e] > m
            m = jnp.where(gt, xs[e], m)
            idx = jnp.where(gt, jnp.full((16,), e, jnp.int32), idx)
        ssum = jnp.zeros((16,), jnp.float32)
        for e in range(E):
            ssum = ssum + jnp.exp(xs[e] - m)
        idxv[pl.ds(g * 16, 16)] = idx
        probv[pl.ds(g * 16, 16)] = 1.0 / ssum

    counts_acc = jnp.zeros((16,), jnp.int32)
    for e in range(E):
        carry = jnp.zeros((16,), jnp.int32)
        for g in range(G):
            ig = idxv[pl.ds(g * 16, 16)]
            hit = ig == e
            mm = hit.astype(jnp.int32)
            cs = plsc.cumsum(mm)
            excl = cs - mm + carry
            lrankv[pl.ds(g * 16, 16)] = jnp.where(
                hit, excl, lrankv[pl.ds(g * 16, 16)])
            carry = carry + jnp.sum(mm)
        counts_acc = jnp.where(lane == e, carry, counts_acc)
    cntv[...] = counts_acc

    pltpu.sync_copy(idxv, idx_hbm.at[pl.ds(t0, CHUNK)])
    pltpu.sync_copy(lrankv, lrank_hbm.at[pl.ds(t0, CHUNK)])
    pltpu.sync_copy(probv, prob_hbm.at[pl.ds(t0, CHUNK)])
    pltpu.sync_copy(cntv, counts_hbm.at[wid])


@functools.cache
def _make_route():
    return functools.partial(
        pl.kernel,
        out_type=[
            jax.ShapeDtypeStruct((S,), jnp.int32),
            jax.ShapeDtypeStruct((S,), jnp.int32),
            jax.ShapeDtypeStruct((S,), jnp.float32),
            jax.ShapeDtypeStruct((NW, 16), jnp.int32),
        ],
        mesh=plsc.VectorSubcoreMesh(
            core_axis_name="c", subcore_axis_name="s",
            num_cores=NC, num_subcores=NS),
        scratch_types=[
            pltpu.VMEM((CHUNK * E,), jnp.float32),
            pltpu.VMEM((CHUNK,), jnp.int32),
            pltpu.VMEM((CHUNK,), jnp.int32),
            pltpu.VMEM((CHUNK,), jnp.float32),
            pltpu.VMEM((16,), jnp.int32),
        ],
        compiler_params=pltpu.CompilerParams(needs_layout_passes=False),
    )(_route_body)


TB = CHUNK


def _expand_body(idx_ref, lrank_ref, prob_ref, counts_ref, comb_ref, mask_ref):
    i = pl.program_id(0)
    idx = idx_ref[...]
    lr = lrank_ref[...]
    pb = prob_ref[...]
    cnt = counts_ref[...]

    rowi = lax.broadcasted_iota(jnp.int32, (NW, 16), 0)
    off_row = jnp.sum(jnp.where(rowi < i, cnt, 0), axis=0, keepdims=True)
    eio = lax.broadcasted_iota(jnp.int32, (TB, 16), 1)
    off = jnp.sum(jnp.where(idx == eio, off_row, 0), axis=1, keepdims=True)

    rank = lr + off
    keep = rank < CAP
    pos = jnp.where(keep, idx * CAP + rank, -1)
    col = lax.broadcasted_iota(jnp.int32, (TB, F), 1)
    hit = col == pos
    comb_ref[...] = jnp.where(hit, pb, 0.0)
    mask_ref[...] = hit


def kernel(inputs):
    idx, lrank, prob, counts = _make_route()(inputs.reshape(S * E))
    comb, mask = pl.pallas_call(
        _expand_body,
        grid=(S // TB,),
        in_specs=[
            pl.BlockSpec((TB, 1), lambda i: (i, 0)),
            pl.BlockSpec((TB, 1), lambda i: (i, 0)),
            pl.BlockSpec((TB, 1), lambda i: (i, 0)),
            pl.BlockSpec((NW, 16), lambda i: (0, 0)),
        ],
        out_specs=[
            pl.BlockSpec((TB, F), lambda i: (i, 0)),
            pl.BlockSpec((TB, F), lambda i: (i, 0)),
        ],
        out_shape=[
            jax.ShapeDtypeStruct((S, F), jnp.float32),
            jax.ShapeDtypeStruct((S, F), jnp.bool_),
        ],
    )(idx.reshape(S, 1), lrank.reshape(S, 1), prob.reshape(S, 1), counts)
    return comb.reshape(S, E, CAP), mask.reshape(S, E, CAP)


# speedup vs baseline: 1.6811x; 1.6811x over previous
import functools
import math

import jax
import jax.numpy as jnp
from jax import lax
from jax.experimental import pallas as pl
from jax.experimental.pallas import tpu as pltpu
from jax.experimental.pallas import tpu_sc as plsc

S = 4096
E = 8
NC = 2
NS = 16
NW = NC * NS
CHUNK = S // NW
G = CHUNK // 16


def _cap(s, e):
    c = math.floor(1.0 * s / e)
    c += c % 2
    return max(c, 4)


CAP = _cap(S, E)
F = E * CAP


def _route_body(x_hbm, idx_hbm, lrank_hbm, prob_hbm, counts_hbm,
                xv, idxv, lrankv, probv, cntv):
    c = lax.axis_index("c")
    s = lax.axis_index("s")
    wid = s * NC + c
    t0 = wid * CHUNK
    pltpu.sync_copy(x_hbm.at[pl.ds(t0 * E, CHUNK * E)], xv)

    lane = jnp.arange(16, dtype=jnp.int32)
    for g in range(G):
        base = (lane + g * 16) * E
        xs = [plsc.load_gather(xv, [base + e]) for e in range(E)]
        m = xs[0]
        idx = jnp.zeros((16,), jnp.int32)
        for e in range(1, E):
            gt = xs[e] > m
            m = jnp.where(gt, xs[e], m)
            idx = jnp.where(gt, jnp.full((16,), e, jnp.int32), idx)
        ssum = jnp.zeros((16,), jnp.float32)
        for e in range(E):
            ssum = ssum + jnp.exp(xs[e] - m)
        idxv[pl.ds(g * 16, 16)] = idx
        probv[pl.ds(g * 16, 16)] = 1.0 / ssum

    counts_acc = jnp.zeros((16,), jnp.int32)
    for e in range(E):
        carry = jnp.zeros((16,), jnp.int32)
        for g in range(G):
            ig = idxv[pl.ds(g * 16, 16)]
            hit = ig == e
            mm = hit.astype(jnp.int32)
            cs = plsc.cumsum(mm)
            excl = cs - mm + carry
            lrankv[pl.ds(g * 16, 16)] = jnp.where(
                hit, excl, lrankv[pl.ds(g * 16, 16)])
            carry = carry + jnp.sum(mm)
        counts_acc = jnp.where(lane == e, carry, counts_acc)
    cntv[...] = counts_acc

    pltpu.sync_copy(idxv, idx_hbm.at[pl.ds(t0, CHUNK)])
    pltpu.sync_copy(lrankv, lrank_hbm.at[pl.ds(t0, CHUNK)])
    pltpu.sync_copy(probv, prob_hbm.at[pl.ds(t0, CHUNK)])
    pltpu.sync_copy(cntv, counts_hbm.at[wid])


@functools.cache
def _make_route():
    return functools.partial(
        pl.kernel,
        out_type=[
            jax.ShapeDtypeStruct((S,), jnp.int32),
            jax.ShapeDtypeStruct((S,), jnp.int32),
            jax.ShapeDtypeStruct((S,), jnp.float32),
            jax.ShapeDtypeStruct((NW, 16), jnp.int32),
        ],
        mesh=plsc.VectorSubcoreMesh(
            core_axis_name="c", subcore_axis_name="s",
            num_cores=NC, num_subcores=NS),
        scratch_types=[
            pltpu.VMEM((CHUNK * E,), jnp.float32),
            pltpu.VMEM((CHUNK,), jnp.int32),
            pltpu.VMEM((CHUNK,), jnp.int32),
            pltpu.VMEM((CHUNK,), jnp.float32),
            pltpu.VMEM((16,), jnp.int32),
        ],
        compiler_params=pltpu.CompilerParams(needs_layout_passes=False),
    )(_route_body)


TB = CHUNK


def _expand_body(idx_ref, lrank_ref, prob_ref, counts_ref, comb_ref, mask_ref):
    i = pl.program_id(0)
    idx = idx_ref[...]
    lr = lrank_ref[...]
    pb = prob_ref[...]
    cnt = counts_ref[...]

    rowi = lax.broadcasted_iota(jnp.int32, (NW, 16), 0)
    off_row = jnp.sum(jnp.where(rowi < i, cnt, 0), axis=0, keepdims=True)
    eio = lax.broadcasted_iota(jnp.int32, (TB, 16), 1)
    off = jnp.sum(jnp.where(idx == eio, off_row, 0), axis=1, keepdims=True)

    rank = lr + off
    keep = rank < CAP
    pos = jnp.where(keep, idx * CAP + rank, -1).reshape(TB, 1, 1)
    col = (lax.broadcasted_iota(jnp.int32, (TB, E, CAP), 1) * CAP
           + lax.broadcasted_iota(jnp.int32, (TB, E, CAP), 2))
    hit = col == pos
    comb_ref[...] = jnp.where(hit, pb.reshape(TB, 1, 1), 0.0)
    mask_ref[...] = hit


def kernel(inputs):
    idx, lrank, prob, counts = _make_route()(inputs.reshape(S * E))
    comb, mask = pl.pallas_call(
        _expand_body,
        grid=(S // TB,),
        in_specs=[
            pl.BlockSpec((TB, 1), lambda i: (i, 0)),
            pl.BlockSpec((TB, 1), lambda i: (i, 0)),
            pl.BlockSpec((TB, 1), lambda i: (i, 0)),
            pl.BlockSpec((NW, 16), lambda i: (0, 0)),
        ],
        out_specs=[
            pl.BlockSpec((TB, E, CAP), lambda i: (i, 0, 0)),
            pl.BlockSpec((TB, E, CAP), lambda i: (i, 0, 0)),
        ],
        out_shape=[
            jax.ShapeDtypeStruct((S, E, CAP), jnp.float32),
            jax.ShapeDtypeStruct((S, E, CAP), jnp.bool_),
        ],
    )(idx.reshape(S, 1), lrank.reshape(S, 1), prob.reshape(S, 1), counts)
    return comb, mask


# speedup vs baseline: 2.2983x; 1.3671x over previous
import functools
import math

import jax
import jax.numpy as jnp
from jax import lax
from jax.experimental import pallas as pl
from jax.experimental.pallas import tpu as pltpu
from jax.experimental.pallas import tpu_sc as plsc

S = 4096
E = 8
NC = 2
NS = 16
NW = NC * NS
CHUNK = S // NW
G = CHUNK // 16


def _cap(s, e):
    c = math.floor(1.0 * s / e)
    c += c % 2
    return max(c, 4)


CAP = _cap(S, E)


def _route_body(x_hbm, meta_hbm, prob_hbm, counts_hbm, xv, metav, probv, cntv):
    c = lax.axis_index("c")
    s = lax.axis_index("s")
    wid = s * NC + c
    t0 = wid * CHUNK
    for e in range(E):
        pltpu.sync_copy(x_hbm.at[pl.ds(e * S + t0, CHUNK)],
                        xv.at[pl.ds(e * CHUNK, CHUNK)])

    lane = jnp.arange(16, dtype=jnp.int32)
    idxs = []
    for g in range(G):
        xs = [xv[pl.ds(e * CHUNK + g * 16, 16)] for e in range(E)]
        m = xs[0]
        idx = jnp.zeros((16,), jnp.int32)
        for e in range(1, E):
            gt = xs[e] > m
            m = jnp.where(gt, xs[e], m)
            idx = jnp.where(gt, jnp.full((16,), e, jnp.int32), idx)
        ssum = jnp.zeros((16,), jnp.float32)
        for e in range(E):
            ssum = ssum + jnp.exp(xs[e] - m)
        idxs.append(idx)
        probv[pl.ds(g * 16, 16)] = 1.0 / ssum

    counts_acc = jnp.zeros((16,), jnp.int32)
    for e in range(E):
        carry = jnp.zeros((16,), jnp.int32)
        for g in range(G):
            hit = idxs[g] == e
            mm = hit.astype(jnp.int32)
            cs = plsc.cumsum(mm)
            excl = cs - mm + carry
            if e == 0:
                metav[pl.ds(g * 16, 16)] = excl
            else:
                metav[pl.ds(g * 16, 16)] = jnp.where(
                    hit, idxs[g] * 65536 + excl, metav[pl.ds(g * 16, 16)])
            carry = carry + jnp.sum(mm)
        counts_acc = jnp.where(lane == e, carry, counts_acc)
    cntv[...] = counts_acc

    pltpu.sync_copy(metav, meta_hbm.at[pl.ds(t0, CHUNK)])
    pltpu.sync_copy(probv, prob_hbm.at[pl.ds(t0, CHUNK)])
    pltpu.sync_copy(cntv, counts_hbm.at[wid])


@functools.cache
def _make_route():
    return functools.partial(
        pl.kernel,
        out_type=[
            jax.ShapeDtypeStruct((S,), jnp.int32),
            jax.ShapeDtypeStruct((S,), jnp.float32),
            jax.ShapeDtypeStruct((NW, 16), jnp.int32),
        ],
        mesh=plsc.VectorSubcoreMesh(
            core_axis_name="c", subcore_axis_name="s",
            num_cores=NC, num_subcores=NS),
        scratch_types=[
            pltpu.VMEM((CHUNK * E,), jnp.float32),
            pltpu.VMEM((CHUNK,), jnp.int32),
            pltpu.VMEM((CHUNK,), jnp.float32),
            pltpu.VMEM((16,), jnp.int32),
        ],
        compiler_params=pltpu.CompilerParams(needs_layout_passes=False),
    )(_route_body)


TB = CHUNK


def _expand_body(meta_ref, prob_ref, counts_ref, comb_ref, mask_ref):
    i = pl.program_id(0)
    meta = meta_ref[...].reshape(TB, 1)
    pb = prob_ref[...].reshape(TB, 1)
    cnt = counts_ref[...]
    idx = meta >> 16
    lr = meta & 65535

    rowi = lax.broadcasted_iota(jnp.int32, (NW, 16), 0)
    off_row = jnp.sum(jnp.where(rowi < i, cnt, 0), axis=0, keepdims=True)
    eio = lax.broadcasted_iota(jnp.int32, (TB, 16), 1)
    off = jnp.sum(jnp.where(idx == eio, off_row, 0), axis=1, keepdims=True)

    rank = lr + off
    keep = rank < CAP
    pos = jnp.where(keep, idx * CAP + rank, -1).reshape(TB, 1, 1)
    col = (lax.broadcasted_iota(jnp.int32, (TB, E, CAP), 1) * CAP
           + lax.broadcasted_iota(jnp.int32, (TB, E, CAP), 2))
    hit = col == pos
    comb_ref[...] = jnp.where(hit, pb.reshape(TB, 1, 1), 0.0)
    mask_ref[...] = hit.astype(jnp.int8)


def kernel(inputs):
    xT = inputs.T.reshape(S * E)
    meta, prob, counts = _make_route()(xT)
    comb, mask = pl.pallas_call(
        _expand_body,
        grid=(S // TB,),
        in_specs=[
            pl.BlockSpec((TB,), lambda i: (i,)),
            pl.BlockSpec((TB,), lambda i: (i,)),
            pl.BlockSpec((NW, 16), lambda i: (0, 0)),
        ],
        out_specs=[
            pl.BlockSpec((TB, E, CAP), lambda i: (i, 0, 0)),
            pl.BlockSpec((TB, E, CAP), lambda i: (i, 0, 0)),
        ],
        out_shape=[
            jax.ShapeDtypeStruct((S, E, CAP), jnp.float32),
            jax.ShapeDtypeStruct((S, E, CAP), jnp.int8),
        ],
    )(meta, prob, counts)
    return comb, mask.astype(jnp.bool_)
